# baseline (speedup 1.0000x reference)
import functools

import jax
import jax.numpy as jnp
from jax import lax
from jax.experimental import pallas as pl
from jax.experimental.pallas import tpu as pltpu
from jax.experimental.pallas import tpu_sc as plsc

_BATCH = 16384
_DIM = 64
_PAD_DIM = 128
_TABLE_ROWS = 100000
_GAMMA = 12.0

_NC = 2
_NS = 16
_L = 16
_NW = _NC * _NS
_ROWS_PER_W = _BATCH // _NW
_CHUNK = 128
_NCHUNK = _ROWS_PER_W // _CHUNK

_TB = 8192
_TG = (_TABLE_ROWS + _TB - 1) // _TB


def _transpose_body(ent_ref, rel_ref, ent_o, rel_o):
    ent_o[:, :_DIM] = jnp.swapaxes(ent_ref[...], 0, 1)
    rel_o[:, :_DIM] = jnp.swapaxes(rel_ref[...], 0, 1)


_transpose_tables = pl.pallas_call(
    _transpose_body,
    grid=(_TG,),
    in_specs=[
        pl.BlockSpec((_DIM, _TB), lambda i: (0, i)),
        pl.BlockSpec((_DIM, _TB), lambda i: (0, i)),
    ],
    out_specs=[
        pl.BlockSpec((_TB, _PAD_DIM), lambda i: (i, 0)),
        pl.BlockSpec((_TB, _PAD_DIM), lambda i: (i, 0)),
    ],
    out_shape=(
        jax.ShapeDtypeStruct((_TABLE_ROWS, _PAD_DIM), jnp.float32),
        jax.ShapeDtypeStruct((_TABLE_ROWS, _PAD_DIM), jnp.float32),
    ),
)


def _compute_chunk(rows_h, rows_r, rows_t, out_v, out_base):
    lane = lax.iota(jnp.int32, _L)

    def block_body(b, carry):
        acc = jnp.zeros((_L,), jnp.float32)
        for l in range(_L):
            row = b * _L + l
            psum = jnp.zeros((_L,), jnp.float32)
            for j in range(_DIM // _L):
                sl = pl.ds(j * _L, _L)
                hv = rows_h[row, sl]
                rv = rows_r[row, sl]
                tv = rows_t[row, sl]
                psum = psum + jnp.abs(hv + rv - tv)
            total = jnp.sum(psum) - _GAMMA
            acc = jnp.where(lane == l, total, acc)
        out_v[pl.ds(out_base + b * _L, _L)] = acc
        return carry

    lax.fori_loop(0, _CHUNK // _L, block_body, 0)


def _body(hidx_hbm, ridx_hbm, tidx_hbm, ent_hbm, rel_hbm, out_hbm,
          idx_h, idx_r, idx_t,
          rows_ha, rows_ra, rows_ta, rows_hb, rows_rb, rows_tb,
          out_v, sem_a, sem_b):
    wid = lax.axis_index("s") * _NC + lax.axis_index("c")
    base = wid * _ROWS_PER_W

    for c in range(_NCHUNK):
        src = pl.ds(base + c * _CHUNK, _CHUNK)
        pltpu.sync_copy(hidx_hbm.at[src], idx_h.at[c])
        pltpu.sync_copy(ridx_hbm.at[src], idx_r.at[c])
        pltpu.sync_copy(tidx_hbm.at[src], idx_t.at[c])

    bufs = ((rows_ha, rows_ra, rows_ta), (rows_hb, rows_rb, rows_tb))
    sems = (sem_a, sem_b)

    def fire(c, bset, sem):
        return (
            pltpu.async_copy(ent_hbm.at[idx_h.at[c]], bset[0], sem),
            pltpu.async_copy(rel_hbm.at[idx_r.at[c]], bset[1], sem),
            pltpu.async_copy(ent_hbm.at[idx_t.at[c]], bset[2], sem),
        )

    cps = fire(0, bufs[0], sems[0])
    for c in range(_NCHUNK):
        nxt = None
        if c + 1 < _NCHUNK:
            nxt = fire(c + 1, bufs[(c + 1) % 2], sems[(c + 1) % 2])
        for cp in cps:
            cp.wait()
        _compute_chunk(*bufs[c % 2], out_v, c * _CHUNK)
        cps = nxt

    pltpu.sync_copy(out_v, out_hbm.at[pl.ds(base, _ROWS_PER_W)])


@functools.partial(
    pl.kernel,
    out_type=jax.ShapeDtypeStruct((_BATCH,), jnp.float32),
    scratch_types=[
        pltpu.VMEM((_NCHUNK, _CHUNK), jnp.int32),
        pltpu.VMEM((_NCHUNK, _CHUNK), jnp.int32),
        pltpu.VMEM((_NCHUNK, _CHUNK), jnp.int32),
        pltpu.VMEM((_CHUNK, _PAD_DIM), jnp.float32),
        pltpu.VMEM((_CHUNK, _PAD_DIM), jnp.float32),
        pltpu.VMEM((_CHUNK, _PAD_DIM), jnp.float32),
        pltpu.VMEM((_CHUNK, _PAD_DIM), jnp.float32),
        pltpu.VMEM((_CHUNK, _PAD_DIM), jnp.float32),
        pltpu.VMEM((_CHUNK, _PAD_DIM), jnp.float32),
        pltpu.VMEM((_ROWS_PER_W,), jnp.float32),
        pltpu.SemaphoreType.DMA,
        pltpu.SemaphoreType.DMA,
    ],
    mesh=plsc.VectorSubcoreMesh(core_axis_name="c", subcore_axis_name="s"),
    compiler_params=pltpu.CompilerParams(
        needs_layout_passes=False, use_tc_tiling_on_sc=False
    ),
)
def _transe_sc(*args):
    _body(*args)


def kernel(pos_sample, ent_embd, rel_embd):
    ent_p, rel_p = _transpose_tables(ent_embd.T, rel_embd.T)
    h_idx = pos_sample[:, 0]
    r_idx = pos_sample[:, 1]
    t_idx = pos_sample[:, 2]
    score = _transe_sc(h_idx, r_idx, t_idx, ent_p, rel_p)
    return score[:, None]
